# baseline (speedup 1.0000x reference)
import functools

import jax
import jax.numpy as jnp
from jax import lax
from jax.experimental import pallas as pl
from jax.experimental.pallas import tpu as pltpu
from jax.experimental.pallas import tpu_sc as plsc

NC = 2
NS = 16
NW = NC * NS
LANES = 16
BM = 1000

F32 = jnp.float32


def _sc_degree(src, n):
    e = src.shape[0]
    epw = e // NW
    mesh = plsc.VectorSubcoreMesh(core_axis_name="c", subcore_axis_name="s")

    @functools.partial(
        pl.kernel,
        out_type=jax.ShapeDtypeStruct((NW, n), F32),
        mesh=mesh,
        scratch_types=[
            pltpu.VMEM((epw,), jnp.int32),
            pltpu.VMEM((n,), F32),
        ],
        compiler_params=pltpu.CompilerParams(needs_layout_passes=False, skip_device_barrier=True),
    )
    def deg_kernel(src_hbm, out_hbm, src_v, deg_v):
        c = lax.axis_index("c")
        s = lax.axis_index("s")
        w = s * NC + c
        pltpu.sync_copy(src_hbm.at[pl.ds(w * epw, epw)], src_v)
        z16 = jnp.zeros((LANES,), F32)
        ones16 = jnp.ones((LANES,), F32)

        def zero_body(i, carry):
            deg_v[pl.ds(i * LANES, LANES)] = z16
            return carry

        lax.fori_loop(0, n // LANES, zero_body, 0)

        def count_body(i, carry):
            idx = src_v[pl.ds(i * LANES, LANES)]
            plsc.addupdate_scatter(deg_v, [idx], ones16)
            return carry

        lax.fori_loop(0, epw // LANES, count_body, 0)
        pltpu.sync_copy(deg_v, out_hbm.at[w])

    return deg_kernel(src)


def _sc_propagate(q, src, dst, n):
    e = src.shape[0]
    d = q.shape[1]
    epw = e // NW
    K = 40
    nchunks = epw // K
    NB = 6
    ngroups = nchunks // NB
    tail = nchunks - ngroups * NB
    RPT = (n // NS) // 8 * 8
    RPT_LAST = n - (NS - 1) * RPT
    ZR = 8
    mesh = plsc.VectorSubcoreMesh(core_axis_name="c", subcore_axis_name="s")

    scratch = ([pltpu.VMEM((epw,), jnp.int32) for _ in range(2)] +
               [pltpu.VMEM((K, d), F32) for _ in range(NB)] +
               [pltpu.VMEM_SHARED((n, d), F32)] +
               [pltpu.SemaphoreType.DMA for _ in range(2 * NB)])

    @functools.partial(
        pl.kernel,
        out_type=jax.ShapeDtypeStruct((NC, n, d), F32),
        mesh=mesh,
        scratch_types=scratch,
        compiler_params=pltpu.CompilerParams(needs_layout_passes=False, skip_device_barrier=True),
    )
    def prop_kernel(q_hbm, src_hbm, dst_hbm, out_hbm, *scr):
        sidx_all = scr[0]
        didx_all = scr[1]
        rows = scr[2:2 + NB]
        acc = scr[2 + NB]
        gsem = scr[3 + NB:3 + 2 * NB]
        ssem = scr[3 + 2 * NB:3 + 3 * NB]

        c = lax.axis_index("c")
        s = lax.axis_index("s")
        w = s * NC + c
        z16 = jnp.zeros((LANES,), F32)

        pltpu.sync_copy(src_hbm.at[pl.ds(w * epw, epw)], sidx_all)
        pltpu.sync_copy(dst_hbm.at[pl.ds(w * epw, epw)], didx_all)

        def zb(i, carry):
            for j in range(d // LANES):
                rows[0][i, pl.ds(j * LANES, LANES)] = z16
            return carry

        lax.fori_loop(0, K, zb, 0)
        r0 = s * RPT
        nz = jnp.where(s == NS - 1, RPT_LAST // ZR, RPT // ZR)

        def zcopy(t, carry):
            pltpu.async_copy(rows[0].at[pl.ds(0, ZR)],
                             acc.at[pl.ds(r0 + t * ZR, ZR)], gsem[0])
            return carry

        lax.fori_loop(0, nz, zcopy, 0)

        def zdrain(t, carry):
            pltpu.make_async_copy(rows[0].at[pl.ds(0, ZR)],
                                  acc.at[pl.ds(r0, ZR)], gsem[0]).wait()
            return carry

        lax.fori_loop(0, nz, zdrain, 0)
        plsc.subcore_barrier()

        def start_gather(ch, b):
            pltpu.async_copy(q_hbm.at[sidx_all.at[pl.ds(ch * K, K)]],
                             rows[b], gsem[b])

        def wait_gather_start_scatter(ch, b):
            pltpu.make_async_copy(q_hbm.at[sidx_all.at[pl.ds(ch * K, K)]],
                                  rows[b], gsem[b]).wait()
            pltpu.async_copy(rows[b], acc.at[didx_all.at[pl.ds(ch * K, K)]],
                             ssem[b], add=True)

        def wait_scatter(ch, b):
            pltpu.make_async_copy(rows[b],
                                  acc.at[didx_all.at[pl.ds(ch * K, K)]],
                                  ssem[b]).wait()

        def group(g, carry):
            for b in range(NB):
                @pl.when(g > 0)
                def _():
                    wait_scatter(0, b)
                start_gather(g * NB + b, b)
            for b in range(NB):
                wait_gather_start_scatter(g * NB + b, b)
            return carry

        lax.fori_loop(0, ngroups, group, 0)

        for b in range(tail):
            wait_scatter(0, b)
            start_gather(ngroups * NB + b, b)
        for b in range(tail):
            wait_gather_start_scatter(ngroups * NB + b, b)
        for b in range(NB):
            wait_scatter(0, b)

        plsc.subcore_barrier()

        @pl.when(s < NS - 1)
        def _():
            pltpu.sync_copy(acc.at[pl.ds(r0, RPT)],
                            out_hbm.at[c, pl.ds(r0, RPT)])

        @pl.when(s == NS - 1)
        def _():
            pltpu.sync_copy(acc.at[pl.ds(r0, RPT_LAST)],
                            out_hbm.at[c, pl.ds(r0, RPT_LAST)])

    return prop_kernel(q, src, dst)


def _tc_prep(deg_parts, x, w0, w1):
    n, d = x.shape
    nb = n // BM

    def body(degp_ref, x_ref, w0_ref, w1_ref, dis_ref, a_ref, q_ref):
        deg = jnp.sum(degp_ref[...], axis=1)
        dis = jnp.where(deg > 0, 1.0 / jnp.sqrt(jnp.maximum(deg, 1e-12)), 0.0)
        dis_ref[...] = dis[:, None]
        xb = x_ref[...]
        a_ref[...] = jnp.dot(xb, w0_ref[...], preferred_element_type=F32)
        q_ref[...] = jnp.dot(xb * dis[:, None], w1_ref[...],
                             preferred_element_type=F32)

    return pl.pallas_call(
        body,
        grid=(nb,),
        in_specs=[
            pl.BlockSpec((BM, NW), lambda i: (i, 0)),
            pl.BlockSpec((BM, d), lambda i: (i, 0)),
            pl.BlockSpec(w0.shape, lambda i: (0, 0)),
            pl.BlockSpec(w1.shape, lambda i: (0, 0)),
        ],
        out_specs=[
            pl.BlockSpec((BM, 1), lambda i: (i, 0)),
            pl.BlockSpec((BM, w0.shape[1]), lambda i: (i, 0)),
            pl.BlockSpec((BM, w1.shape[1]), lambda i: (i, 0)),
        ],
        out_shape=[
            jax.ShapeDtypeStruct((n, 1), F32),
            jax.ShapeDtypeStruct((n, w0.shape[1]), F32),
            jax.ShapeDtypeStruct((n, w1.shape[1]), F32),
        ],
    )(deg_parts, x, w0, w1)


def _bn_from_stats(ps_ref, z, gamma, beta, n):
    m = ps_ref[0, :] / n
    v = jnp.maximum(ps_ref[1, :] / n - m * m, 0.0)
    inv = 1.0 / jnp.sqrt(v + 1e-5)
    return (z - m[None, :]) * inv[None, :] * gamma + beta


def _acc_stats(ps_ref, z, first):
    p = jnp.stack([jnp.sum(z, axis=0), jnp.sum(z * z, axis=0)])

    @pl.when(first)
    def _():
        ps_ref[...] = p

    @pl.when(jnp.logical_not(first))
    def _():
        ps_ref[...] = ps_ref[...] + p


def _tc_mid(a, r, dis, gamma, beta, w0n, w1n):
    n, d = a.shape
    nb = n // BM

    def body(a_ref, r_ref, dis_ref, g_ref, b_ref, w0_ref, w1_ref,
             a2_ref, q2_ref, z_scr, ps_scr):
        i = pl.program_id(0)

        @pl.when(i < nb)
        def _():
            z = a_ref[...] - dis_ref[...] * (r_ref[0] + r_ref[1])
            z_scr[pl.ds(lax.rem(i, nb) * BM, BM), :] = z
            _acc_stats(ps_scr, z, i == 0)

        @pl.when(i >= nb)
        def _():
            j = lax.rem(i, nb)
            z = z_scr[pl.ds(j * BM, BM), :]
            h = jnp.maximum(
                _bn_from_stats(ps_scr, z, g_ref[...], b_ref[...], n), 0.0)
            a2_ref[...] = jnp.dot(h, w0_ref[...], preferred_element_type=F32)
            q2_ref[...] = jnp.dot(h * dis_ref[...], w1_ref[...],
                                  preferred_element_type=F32)

    ph = lambda i: (lax.rem(i, nb), 0)
    ph0 = lambda i: (jnp.where(i < nb, i, 0), 0)

    return pl.pallas_call(
        body,
        grid=(2 * nb,),
        in_specs=[
            pl.BlockSpec((BM, d), ph0),
            pl.BlockSpec((NC, BM, d), lambda i: (0, jnp.where(i < nb, i, 0), 0)),
            pl.BlockSpec((BM, 1), ph),
            pl.BlockSpec(gamma.shape, lambda i: (0, 0)),
            pl.BlockSpec(beta.shape, lambda i: (0, 0)),
            pl.BlockSpec(w0n.shape, lambda i: (0, 0)),
            pl.BlockSpec(w1n.shape, lambda i: (0, 0)),
        ],
        out_specs=[
            pl.BlockSpec((BM, w0n.shape[1]), lambda i: (jnp.maximum(i - nb, 0), 0)),
            pl.BlockSpec((BM, w1n.shape[1]), lambda i: (jnp.maximum(i - nb, 0), 0)),
        ],
        out_shape=[
            jax.ShapeDtypeStruct((n, w0n.shape[1]), F32),
            jax.ShapeDtypeStruct((n, w1n.shape[1]), F32),
        ],
        scratch_shapes=[pltpu.VMEM((n, d), F32), pltpu.VMEM((2, d), F32)],
    )(a, r, dis, gamma, beta, w0n, w1n)


def _tc_final(a, r, dis, gamma2, beta2, wf1, gamma3, beta3, wf2, bf2):
    n, d = a.shape
    dm = wf1.shape[1]
    do = wf2.shape[1]
    nb = n // BM

    def body(a_ref, r_ref, dis_ref, g2_ref, b2_ref, w1_ref, g3_ref, b3_ref,
             w2_ref, bf_ref, out_ref, mean_ref, z_scr, g_scr, ps2, ps3, hs):
        i = pl.program_id(0)
        j = lax.rem(i, nb)

        @pl.when(i < nb)
        def _():
            z = a_ref[...] - dis_ref[...] * (r_ref[0] + r_ref[1])
            z_scr[pl.ds(j * BM, BM), :] = z
            _acc_stats(ps2, z, i == 0)

        @pl.when(jnp.logical_and(i >= nb, i < 2 * nb))
        def _():
            z = z_scr[pl.ds(j * BM, BM), :]
            h2 = jnp.maximum(
                _bn_from_stats(ps2, z, g2_ref[...], b2_ref[...], n), 0.0)
            gm = jnp.dot(h2, w1_ref[...], preferred_element_type=F32)
            g_scr[pl.ds(j * BM, BM), :] = gm
            _acc_stats(ps3, gm, i == nb)

        @pl.when(i >= 2 * nb)
        def _():
            gm = g_scr[pl.ds(j * BM, BM), :]
            h3 = jnp.maximum(
                _bn_from_stats(ps3, gm, g3_ref[...], b3_ref[...], n), 0.0)
            out_ref[...] = jnp.dot(h3, w2_ref[...],
                                   preferred_element_type=F32) + bf_ref[...]
            bsum = jnp.sum(h3, axis=0)[None]

            @pl.when(i == 2 * nb)
            def _():
                hs[...] = bsum

            @pl.when(i > 2 * nb)
            def _():
                hs[...] = hs[...] + bsum

            @pl.when(i == 3 * nb - 1)
            def _():
                mean_ref[...] = jnp.dot(hs[...] / n, w2_ref[...],
                                        preferred_element_type=F32) + bf_ref[...]

    ph0 = lambda i: (jnp.where(i < nb, i, 0), 0)

    return pl.pallas_call(
        body,
        grid=(3 * nb,),
        in_specs=[
            pl.BlockSpec((BM, d), ph0),
            pl.BlockSpec((NC, BM, d), lambda i: (0, jnp.where(i < nb, i, 0), 0)),
            pl.BlockSpec((BM, 1), ph0),
            pl.BlockSpec(gamma2.shape, lambda i: (0, 0)),
            pl.BlockSpec(beta2.shape, lambda i: (0, 0)),
            pl.BlockSpec(wf1.shape, lambda i: (0, 0)),
            pl.BlockSpec(gamma3.shape, lambda i: (0, 0)),
            pl.BlockSpec(beta3.shape, lambda i: (0, 0)),
            pl.BlockSpec(wf2.shape, lambda i: (0, 0)),
            pl.BlockSpec(bf2.shape, lambda i: (0, 0)),
        ],
        out_specs=[
            pl.BlockSpec((BM, do), lambda i: (jnp.maximum(i - 2 * nb, 0), 0)),
            pl.BlockSpec((1, do), lambda i: (0, 0)),
        ],
        out_shape=[
            jax.ShapeDtypeStruct((n, do), F32),
            jax.ShapeDtypeStruct((1, do), F32),
        ],
        scratch_shapes=[pltpu.VMEM((n, d), F32), pltpu.VMEM((n, dm), F32),
                        pltpu.VMEM((2, d), F32), pltpu.VMEM((2, dm), F32),
                        pltpu.VMEM((1, dm), F32)],
    )(a, r, dis, gamma2, beta2, wf1, gamma3, beta3, wf2, bf2)


def kernel(x, edge_index, W1_0, W1_1, b1, gamma1, beta1, W2_0, W2_1, b2,
           gamma2, beta2, Wf1, bf1, gamma3, beta3, Wf2, bf2):
    n = x.shape[0]
    g1 = gamma1[None, :]
    bt1 = beta1[None, :]
    g2 = gamma2[None, :]
    bt2 = beta2[None, :]
    g3 = gamma3[None, :]
    bt3 = beta3[None, :]
    bf2r = bf2[None, :]

    src = edge_index[0]
    dst = edge_index[1]
    deg_parts = _sc_degree(src, n).T
    dis, a1, q1 = _tc_prep(deg_parts, x, W1_0, W1_1)
    r1 = _sc_propagate(q1, src, dst, n)
    a2, q2 = _tc_mid(a1, r1, dis, g1, bt1, W2_0, W2_1)
    r2 = _sc_propagate(q2, src, dst, n)
    out, x_mean = _tc_final(a2, r2, dis, g2, bt2, Wf1, g3, bt3, Wf2, bf2r)
    return (out, x_mean)


# speedup vs baseline: 1.0676x; 1.0676x over previous
import functools

import jax
import jax.numpy as jnp
from jax import lax
from jax.experimental import pallas as pl
from jax.experimental.pallas import tpu as pltpu
from jax.experimental.pallas import tpu_sc as plsc

NC = 2
NS = 16
NW = NC * NS
LANES = 16
BM = 2000

F32 = jnp.float32


def _sc_degree(src, n):
    e = src.shape[0]
    epw = e // NW
    mesh = plsc.VectorSubcoreMesh(core_axis_name="c", subcore_axis_name="s")

    @functools.partial(
        pl.kernel,
        out_type=jax.ShapeDtypeStruct((NW, n), F32),
        mesh=mesh,
        scratch_types=[
            pltpu.VMEM((epw,), jnp.int32),
            pltpu.VMEM((n,), F32),
        ],
        compiler_params=pltpu.CompilerParams(needs_layout_passes=False),
    )
    def deg_kernel(src_hbm, out_hbm, src_v, deg_v):
        c = lax.axis_index("c")
        s = lax.axis_index("s")
        w = s * NC + c
        pltpu.sync_copy(src_hbm.at[pl.ds(w * epw, epw)], src_v)
        z16 = jnp.zeros((LANES,), F32)
        ones16 = jnp.ones((LANES,), F32)

        UNR = 5

        def zero_body(i, carry):
            for u in range(UNR):
                deg_v[pl.ds((i * UNR + u) * LANES, LANES)] = z16
            return carry

        lax.fori_loop(0, n // (LANES * UNR), zero_body, 0)

        def count_body(i, carry):
            for u in range(UNR):
                idx = src_v[pl.ds((i * UNR + u) * LANES, LANES)]
                plsc.addupdate_scatter(deg_v, [idx], ones16)
            return carry

        lax.fori_loop(0, epw // (LANES * UNR), count_body, 0)
        pltpu.sync_copy(deg_v, out_hbm.at[w])

    return deg_kernel(src)


def _sc_propagate(q, src, dst, n):
    e = src.shape[0]
    d = q.shape[1]
    epw = e // NW
    K = 40
    nchunks = epw // K
    NB = 6
    ngroups = nchunks // NB
    tail = nchunks - ngroups * NB
    RPT = (n // NS) // 8 * 8
    RPT_LAST = n - (NS - 1) * RPT
    ZR = 8
    mesh = plsc.VectorSubcoreMesh(core_axis_name="c", subcore_axis_name="s")

    scratch = ([pltpu.VMEM((epw,), jnp.int32) for _ in range(2)] +
               [pltpu.VMEM((K, d), F32) for _ in range(NB)] +
               [pltpu.VMEM_SHARED((n, d), F32)] +
               [pltpu.SemaphoreType.DMA for _ in range(2 * NB)])

    @functools.partial(
        pl.kernel,
        out_type=jax.ShapeDtypeStruct((NC, n, d), F32),
        mesh=mesh,
        scratch_types=scratch,
        compiler_params=pltpu.CompilerParams(needs_layout_passes=False),
    )
    def prop_kernel(q_hbm, src_hbm, dst_hbm, out_hbm, *scr):
        sidx_all = scr[0]
        didx_all = scr[1]
        rows = scr[2:2 + NB]
        acc = scr[2 + NB]
        gsem = scr[3 + NB:3 + 2 * NB]
        ssem = scr[3 + 2 * NB:3 + 3 * NB]

        c = lax.axis_index("c")
        s = lax.axis_index("s")
        w = s * NC + c
        z16 = jnp.zeros((LANES,), F32)

        pltpu.async_copy(src_hbm.at[pl.ds(w * epw, epw)], sidx_all, ssem[0])
        pltpu.async_copy(dst_hbm.at[pl.ds(w * epw, epw)], didx_all, ssem[0])

        def zb(i, carry):
            for j in range(d // LANES):
                rows[0][i, pl.ds(j * LANES, LANES)] = z16
            return carry

        lax.fori_loop(0, K, zb, 0)
        r0 = s * RPT
        nz = jnp.where(s == NS - 1, RPT_LAST // ZR, RPT // ZR)

        def zcopy(t, carry):
            pltpu.async_copy(rows[0].at[pl.ds(0, ZR)],
                             acc.at[pl.ds(r0 + t * ZR, ZR)], gsem[0])
            return carry

        lax.fori_loop(0, nz, zcopy, 0)

        def zdrain(t, carry):
            pltpu.make_async_copy(rows[0].at[pl.ds(0, ZR)],
                                  acc.at[pl.ds(r0, ZR)], gsem[0]).wait()
            return carry

        lax.fori_loop(0, nz, zdrain, 0)
        pltpu.make_async_copy(src_hbm.at[pl.ds(w * epw, epw)], sidx_all,
                              ssem[0]).wait()
        pltpu.make_async_copy(dst_hbm.at[pl.ds(w * epw, epw)], didx_all,
                              ssem[0]).wait()
        plsc.subcore_barrier()

        def start_gather(ch, b):
            pltpu.async_copy(q_hbm.at[sidx_all.at[pl.ds(ch * K, K)]],
                             rows[b], gsem[b])

        def wait_gather_start_scatter(ch, b):
            pltpu.make_async_copy(q_hbm.at[sidx_all.at[pl.ds(ch * K, K)]],
                                  rows[b], gsem[b]).wait()
            pltpu.async_copy(rows[b], acc.at[didx_all.at[pl.ds(ch * K, K)]],
                             ssem[b], add=True)

        def wait_scatter(ch, b):
            pltpu.make_async_copy(rows[b],
                                  acc.at[didx_all.at[pl.ds(ch * K, K)]],
                                  ssem[b]).wait()

        def group(g, carry):
            for b in range(NB):
                @pl.when(g > 0)
                def _():
                    wait_scatter(0, b)
                start_gather(g * NB + b, b)
            for b in range(NB):
                wait_gather_start_scatter(g * NB + b, b)
            return carry

        lax.fori_loop(0, ngroups, group, 0)

        for b in range(tail):
            wait_scatter(0, b)
            start_gather(ngroups * NB + b, b)
        for b in range(tail):
            wait_gather_start_scatter(ngroups * NB + b, b)
        for b in range(NB):
            wait_scatter(0, b)

        plsc.subcore_barrier()

        @pl.when(s < NS - 1)
        def _():
            pltpu.sync_copy(acc.at[pl.ds(r0, RPT)],
                            out_hbm.at[c, pl.ds(r0, RPT)])

        @pl.when(s == NS - 1)
        def _():
            pltpu.sync_copy(acc.at[pl.ds(r0, RPT_LAST)],
                            out_hbm.at[c, pl.ds(r0, RPT_LAST)])

    return prop_kernel(q, src, dst)


def _tc_prep(deg_parts, x, w0, w1):
    n, d = x.shape
    nb = n // BM

    def body(degp_ref, x_ref, w0_ref, w1_ref, dis_ref, a_ref, q_ref):
        deg = jnp.sum(degp_ref[...], axis=1)
        dis = jnp.where(deg > 0, 1.0 / jnp.sqrt(jnp.maximum(deg, 1e-12)), 0.0)
        dis_ref[...] = dis[:, None]
        xb = x_ref[...]
        a_ref[...] = jnp.dot(xb, w0_ref[...], preferred_element_type=F32)
        q_ref[...] = jnp.dot(xb * dis[:, None], w1_ref[...],
                             preferred_element_type=F32)

    return pl.pallas_call(
        body,
        grid=(nb,),
        in_specs=[
            pl.BlockSpec((BM, NW), lambda i: (i, 0)),
            pl.BlockSpec((BM, d), lambda i: (i, 0)),
            pl.BlockSpec(w0.shape, lambda i: (0, 0)),
            pl.BlockSpec(w1.shape, lambda i: (0, 0)),
        ],
        out_specs=[
            pl.BlockSpec((BM, 1), lambda i: (i, 0)),
            pl.BlockSpec((BM, w0.shape[1]), lambda i: (i, 0)),
            pl.BlockSpec((BM, w1.shape[1]), lambda i: (i, 0)),
        ],
        out_shape=[
            jax.ShapeDtypeStruct((n, 1), F32),
            jax.ShapeDtypeStruct((n, w0.shape[1]), F32),
            jax.ShapeDtypeStruct((n, w1.shape[1]), F32),
        ],
    )(deg_parts, x, w0, w1)


def _bn_from_stats(ps_ref, z, gamma, beta, n):
    m = ps_ref[0, :] / n
    v = jnp.maximum(ps_ref[1, :] / n - m * m, 0.0)
    inv = 1.0 / jnp.sqrt(v + 1e-5)
    return (z - m[None, :]) * inv[None, :] * gamma + beta


def _acc_stats(ps_ref, z, first):
    p = jnp.stack([jnp.sum(z, axis=0), jnp.sum(z * z, axis=0)])

    @pl.when(first)
    def _():
        ps_ref[...] = p

    @pl.when(jnp.logical_not(first))
    def _():
        ps_ref[...] = ps_ref[...] + p


def _tc_mid(a, r, dis, gamma, beta, w0n, w1n):
    n, d = a.shape
    nb = n // BM

    def body(a_ref, r_ref, dis_ref, g_ref, b_ref, w0_ref, w1_ref,
             a2_ref, q2_ref, z_scr, ps_scr):
        i = pl.program_id(0)

        @pl.when(i < nb)
        def _():
            z = a_ref[...] - dis_ref[...] * (r_ref[0] + r_ref[1])
            z_scr[pl.ds(lax.rem(i, nb) * BM, BM), :] = z
            _acc_stats(ps_scr, z, i == 0)

        @pl.when(i >= nb)
        def _():
            j = lax.rem(i, nb)
            z = z_scr[pl.ds(j * BM, BM), :]
            h = jnp.maximum(
                _bn_from_stats(ps_scr, z, g_ref[...], b_ref[...], n), 0.0)
            a2_ref[...] = jnp.dot(h, w0_ref[...], preferred_element_type=F32)
            q2_ref[...] = jnp.dot(h * dis_ref[...], w1_ref[...],
                                  preferred_element_type=F32)

    ph = lambda i: (lax.rem(i, nb), 0)
    ph0 = lambda i: (jnp.where(i < nb, i, 0), 0)

    return pl.pallas_call(
        body,
        grid=(2 * nb,),
        in_specs=[
            pl.BlockSpec((BM, d), ph0),
            pl.BlockSpec((NC, BM, d), lambda i: (0, jnp.where(i < nb, i, 0), 0)),
            pl.BlockSpec((BM, 1), ph),
            pl.BlockSpec(gamma.shape, lambda i: (0, 0)),
            pl.BlockSpec(beta.shape, lambda i: (0, 0)),
            pl.BlockSpec(w0n.shape, lambda i: (0, 0)),
            pl.BlockSpec(w1n.shape, lambda i: (0, 0)),
        ],
        out_specs=[
            pl.BlockSpec((BM, w0n.shape[1]), lambda i: (jnp.maximum(i - nb, 0), 0)),
            pl.BlockSpec((BM, w1n.shape[1]), lambda i: (jnp.maximum(i - nb, 0), 0)),
        ],
        out_shape=[
            jax.ShapeDtypeStruct((n, w0n.shape[1]), F32),
            jax.ShapeDtypeStruct((n, w1n.shape[1]), F32),
        ],
        scratch_shapes=[pltpu.VMEM((n, d), F32), pltpu.VMEM((2, d), F32)],
    )(a, r, dis, gamma, beta, w0n, w1n)


def _tc_final(a, r, dis, gamma2, beta2, wf1, gamma3, beta3, wf2, bf2):
    n, d = a.shape
    dm = wf1.shape[1]
    do = wf2.shape[1]
    nb = n // BM

    def body(a_ref, r_ref, dis_ref, g2_ref, b2_ref, w1_ref, g3_ref, b3_ref,
             w2_ref, bf_ref, out_ref, mean_ref, z_scr, g_scr, ps2, ps3, hs):
        i = pl.program_id(0)
        j = lax.rem(i, nb)

        @pl.when(i < nb)
        def _():
            z = a_ref[...] - dis_ref[...] * (r_ref[0] + r_ref[1])
            z_scr[pl.ds(j * BM, BM), :] = z
            _acc_stats(ps2, z, i == 0)

        @pl.when(jnp.logical_and(i >= nb, i < 2 * nb))
        def _():
            z = z_scr[pl.ds(j * BM, BM), :]
            h2 = jnp.maximum(
                _bn_from_stats(ps2, z, g2_ref[...], b2_ref[...], n), 0.0)
            gm = jnp.dot(h2, w1_ref[...], preferred_element_type=F32)
            g_scr[pl.ds(j * BM, BM), :] = gm
            _acc_stats(ps3, gm, i == nb)

        @pl.when(i >= 2 * nb)
        def _():
            gm = g_scr[pl.ds(j * BM, BM), :]
            h3 = jnp.maximum(
                _bn_from_stats(ps3, gm, g3_ref[...], b3_ref[...], n), 0.0)
            out_ref[...] = jnp.dot(h3, w2_ref[...],
                                   preferred_element_type=F32) + bf_ref[...]
            bsum = jnp.sum(h3, axis=0)[None]

            @pl.when(i == 2 * nb)
            def _():
                hs[...] = bsum

            @pl.when(i > 2 * nb)
            def _():
                hs[...] = hs[...] + bsum

            @pl.when(i == 3 * nb - 1)
            def _():
                mean_ref[...] = jnp.dot(hs[...] / n, w2_ref[...],
                                        preferred_element_type=F32) + bf_ref[...]

    ph0 = lambda i: (jnp.where(i < nb, i, 0), 0)

    return pl.pallas_call(
        body,
        grid=(3 * nb,),
        in_specs=[
            pl.BlockSpec((BM, d), ph0),
            pl.BlockSpec((NC, BM, d), lambda i: (0, jnp.where(i < nb, i, 0), 0)),
            pl.BlockSpec((BM, 1), ph0),
            pl.BlockSpec(gamma2.shape, lambda i: (0, 0)),
            pl.BlockSpec(beta2.shape, lambda i: (0, 0)),
            pl.BlockSpec(wf1.shape, lambda i: (0, 0)),
            pl.BlockSpec(gamma3.shape, lambda i: (0, 0)),
            pl.BlockSpec(beta3.shape, lambda i: (0, 0)),
            pl.BlockSpec(wf2.shape, lambda i: (0, 0)),
            pl.BlockSpec(bf2.shape, lambda i: (0, 0)),
        ],
        out_specs=[
            pl.BlockSpec((BM, do), lambda i: (jnp.maximum(i - 2 * nb, 0), 0)),
            pl.BlockSpec((1, do), lambda i: (0, 0)),
        ],
        out_shape=[
            jax.ShapeDtypeStruct((n, do), F32),
            jax.ShapeDtypeStruct((1, do), F32),
        ],
        scratch_shapes=[pltpu.VMEM((n, d), F32), pltpu.VMEM((n, dm), F32),
                        pltpu.VMEM((2, d), F32), pltpu.VMEM((2, dm), F32),
                        pltpu.VMEM((1, dm), F32)],
    )(a, r, dis, gamma2, beta2, wf1, gamma3, beta3, wf2, bf2)


def kernel(x, edge_index, W1_0, W1_1, b1, gamma1, beta1, W2_0, W2_1, b2,
           gamma2, beta2, Wf1, bf1, gamma3, beta3, Wf2, bf2):
    n = x.shape[0]
    g1 = gamma1[None, :]
    bt1 = beta1[None, :]
    g2 = gamma2[None, :]
    bt2 = beta2[None, :]
    g3 = gamma3[None, :]
    bt3 = beta3[None, :]
    bf2r = bf2[None, :]

    src = edge_index[0]
    dst = edge_index[1]
    deg_parts = _sc_degree(src, n).T
    dis, a1, q1 = _tc_prep(deg_parts, x, W1_0, W1_1)
    r1 = _sc_propagate(q1, src, dst, n)
    a2, q2 = _tc_mid(a1, r1, dis, g1, bt1, W2_0, W2_1)
    r2 = _sc_propagate(q2, src, dst, n)
    out, x_mean = _tc_final(a2, r2, dis, g2, bt2, Wf1, g3, bt3, Wf2, bf2r)
    return (out, x_mean)
